# baseline (speedup 1.0000x reference)
import functools

import jax
import jax.numpy as jnp
from jax import lax
from jax.experimental import pallas as pl
from jax.experimental.pallas import tpu as pltpu
from jax.experimental.pallas import tpu_sc as plsc

N = 10000
E = 320000
D = 128
G = 64
DO = 16

NC = 2
NS = 16
NW = NC * NS
LANE = 128
CH0 = 112
CH1 = 48
TOTCH = NS * (CH0 + CH1)
CHD = TOTCH // NW
PH = 2
CH0P = CH0 // PH
CH1P = CH1 // PH
CHPM = CH0P
PADCH = TOTCH + CHPM - CH1P
NBR = 2
SDEPTH = 8
EP = PADCH * LANE
NP = 10240
RPC = NP // NS
CW = D
BLK = 512
GRID = NP // BLK


def _deg_body(dst2, ones_hbm, zcnt_hbm, out, dstv, onesv, cnt_sh, ssem):
    c = lax.axis_index("c")
    s = lax.axis_index("s")
    w = s * NC + c
    sl = pl.ds(s * RPC, RPC)
    pltpu.sync_copy(zcnt_hbm.at[sl], cnt_sh.at[sl])
    pltpu.sync_copy(ones_hbm, onesv)
    pltpu.sync_copy(dst2.at[pl.ds(w * CHD, CHD)], dstv)
    plsc.subcore_barrier()

    def chunk(j, carry):
        pltpu.async_copy(onesv, cnt_sh.at[dstv.at[j]], ssem, add=True)

        @pl.when(j >= SDEPTH)
        def _():
            pltpu.make_async_copy(onesv, cnt_sh.at[dstv.at[0]], ssem).wait()

        return carry

    lax.fori_loop(0, CHD, chunk, 0)

    def drain(j, carry):
        pltpu.make_async_copy(onesv, cnt_sh.at[dstv.at[0]], ssem).wait()
        return carry

    lax.fori_loop(0, min(SDEPTH, CHD), drain, 0)
    plsc.subcore_barrier()
    pltpu.sync_copy(cnt_sh.at[sl], out.at[c, sl])


@functools.lru_cache(maxsize=None)
def _sc_calls():
    mesh = plsc.VectorSubcoreMesh(
        core_axis_name="c", subcore_axis_name="s",
        num_cores=NC, num_subcores=NS)
    deg_call = pl.kernel(
        _deg_body,
        out_type=jax.ShapeDtypeStruct((NC, NP, CW), jnp.float32),
        mesh=mesh,
        scratch_types=[
            pltpu.VMEM((CHD, LANE), jnp.int32),
            pltpu.VMEM((LANE, CW), jnp.float32),
            pltpu.VMEM_SHARED((NP, CW), jnp.float32),
            pltpu.SemaphoreType.DMA,
        ],
    )
    agg_call = pl.kernel(
        _agg_body,
        out_type=jax.ShapeDtypeStruct((NC, NP, D), jnp.float32),
        mesh=mesh,
        scratch_types=[
            pltpu.VMEM((CHPM, LANE), jnp.int32),
            pltpu.VMEM((CHPM, LANE), jnp.int32),
            pltpu.VMEM((NBR * LANE, D), jnp.float32),
            pltpu.VMEM_SHARED((NP, D), jnp.float32),
            pltpu.SemaphoreType.DMA,
        ],
    )
    return deg_call, agg_call


def _agg_body(hp, srcc, dstc, zrow_hbm, out, srcv, dstv, rows, acc_sh, gsem):
    c = lax.axis_index("c")
    s = lax.axis_index("s")
    sl = pl.ds(s * RPC, RPC)
    chnp = jnp.where(c == 0, CH0P, CH1P)
    start = jnp.where(c == 0, s * CH0, NS * CH0 + s * CH1)

    @pl.when(c == 0)
    def _():
        pltpu.sync_copy(hp.at[sl], acc_sh.at[sl])

    @pl.when(c != 0)
    def _():
        pltpu.sync_copy(zrow_hbm.at[sl], acc_sh.at[sl])

    plsc.subcore_barrier()

    def slot(j):
        return pl.ds((j % NBR) * LANE, LANE)

    for p in range(PH):
        base = start + p * chnp
        pltpu.sync_copy(srcc.at[pl.ds(base, CHPM)], srcv)
        pltpu.sync_copy(dstc.at[pl.ds(base, CHPM)], dstv)

        @pl.when(c == 0)
        def _():
            pltpu.async_copy(hp.at[srcv.at[0]], rows.at[slot(0)], gsem)

            def chunk(j, carry):
                bs = slot(j)
                pltpu.make_async_copy(hp.at[srcv.at[j]], rows.at[bs],
                                      gsem).wait()

                @pl.when(j + 1 < chnp)
                def _():
                    pltpu.async_copy(hp.at[srcv.at[j + 1]],
                                     rows.at[slot(j + 1)], gsem)

                pltpu.sync_copy(rows.at[bs], acc_sh.at[dstv.at[j]], add=True)
                return carry

            lax.fori_loop(0, chnp, chunk, 0)

        @pl.when(c != 0)
        def _():
            def chunk(j, carry):
                bs = slot(j)
                pltpu.async_copy(hp.at[srcv.at[j]], rows.at[bs], gsem).wait()
                pltpu.sync_copy(rows.at[bs], acc_sh.at[dstv.at[j]], add=True)
                return carry

            lax.fori_loop(0, chnp, chunk, 0)

    plsc.subcore_barrier()
    pltpu.sync_copy(acc_sh.at[sl], out.at[c, sl])


def _dinv_from(deg_ref):
    cnt = jnp.max(deg_ref[0] + deg_ref[1], axis=1, keepdims=True)
    return lax.rsqrt(cnt + 1.0)


def _tc1_body(deg_ref, x_ref, w_ref, out_ref):
    dinv = _dinv_from(deg_ref)
    h = jnp.dot(x_ref[...], w_ref[...], preferred_element_type=jnp.float32)
    out_ref[...] = h * dinv


_tc1_call = pl.pallas_call(
    _tc1_body,
    grid=(GRID,),
    in_specs=[
        pl.BlockSpec((NC, BLK, CW), lambda i: (0, i, 0)),
        pl.BlockSpec((BLK, D), lambda i: (i, 0)),
        pl.BlockSpec((D, D), lambda i: (0, 0)),
    ],
    out_specs=pl.BlockSpec((BLK, D), lambda i: (i, 0)),
    out_shape=jax.ShapeDtypeStruct((NP, D), jnp.float32),
)


def _tc2_body(deg_ref, acc_ref, b_ref, w_ref, out_ref):
    dinv = _dinv_from(deg_ref)
    z = jnp.maximum(dinv * (acc_ref[0] + acc_ref[1]) + b_ref[...], 0.0)
    out_ref[...] = jnp.dot(
        z, w_ref[...], preferred_element_type=jnp.float32) * dinv


_tc2_call = pl.pallas_call(
    _tc2_body,
    grid=(GRID,),
    in_specs=[
        pl.BlockSpec((NC, BLK, CW), lambda i: (0, i, 0)),
        pl.BlockSpec((NC, BLK, D), lambda i: (0, i, 0)),
        pl.BlockSpec((1, D), lambda i: (0, 0)),
        pl.BlockSpec((D, D), lambda i: (0, 0)),
    ],
    out_specs=pl.BlockSpec((BLK, D), lambda i: (i, 0)),
    out_shape=jax.ShapeDtypeStruct((NP, D), jnp.float32),
)


def _tc3_body(deg_ref, acc_ref, b_ref, batch_ref, wf_ref, bf_ref, out_ref,
              pooled, cntp):
    i = pl.program_id(0)

    @pl.when(i == 0)
    def _():
        pooled[...] = jnp.zeros_like(pooled)
        cntp[...] = jnp.zeros_like(cntp)

    dinv = _dinv_from(deg_ref)
    z = jnp.maximum(dinv * (acc_ref[0] + acc_ref[1]) + b_ref[...], 0.0)
    oh = (batch_ref[...] == lax.broadcasted_iota(
        jnp.int32, (BLK, G), 1)).astype(jnp.float32)
    dn = (((0,), (0,)), ((), ()))
    pooled[...] += lax.dot_general(oh, z, dn,
                                   preferred_element_type=jnp.float32)
    cntp[...] += lax.dot_general(oh, jnp.ones((BLK, D), jnp.float32), dn,
                                 preferred_element_type=jnp.float32)
    pool = pooled[...] / jnp.maximum(cntp[...], 1.0)
    out_ref[...] = jnp.dot(
        pool, wf_ref[...], preferred_element_type=jnp.float32) + bf_ref[...]


_tc3_call = pl.pallas_call(
    _tc3_body,
    grid=(GRID,),
    in_specs=[
        pl.BlockSpec((NC, BLK, CW), lambda i: (0, i, 0)),
        pl.BlockSpec((NC, BLK, D), lambda i: (0, i, 0)),
        pl.BlockSpec((1, D), lambda i: (0, 0)),
        pl.BlockSpec((BLK, 1), lambda i: (i, 0)),
        pl.BlockSpec((D, DO), lambda i: (0, 0)),
        pl.BlockSpec((1, DO), lambda i: (0, 0)),
    ],
    out_specs=pl.BlockSpec((G, DO), lambda i: (0, 0)),
    out_shape=jax.ShapeDtypeStruct((G, DO), jnp.float32),
    scratch_shapes=[
        pltpu.VMEM((G, D), jnp.float32),
        pltpu.VMEM((G, D), jnp.float32),
    ],
)


def kernel(x, edge_index, batch, W1, b1, W2, b2, Wf, bf):
    i32 = jnp.int32
    pad_idx = jnp.full((EP - E,), N, i32)
    srcc = jnp.concatenate([edge_index[0].astype(i32), pad_idx])
    srcc = srcc.reshape(PADCH, LANE)
    dstc = jnp.concatenate([edge_index[1].astype(i32), pad_idx])
    dstc = dstc.reshape(PADCH, LANE)
    xp = jnp.pad(x.astype(jnp.float32), ((0, NP - N), (0, 0)))
    batchp = jnp.pad(batch.astype(i32), (0, NP - N),
                     constant_values=G).reshape(NP, 1)
    ones_cw = jnp.ones((LANE, CW), jnp.float32)
    zrow = jnp.zeros((NP, D), jnp.float32)

    _deg_call, _agg_call = _sc_calls()
    deg = _deg_call(dstc, ones_cw, zrow)
    hp1 = _tc1_call(deg, xp, W1.astype(jnp.float32))
    acc1 = _agg_call(hp1, srcc, dstc, zrow)
    hp2 = _tc2_call(deg, acc1, b1.reshape(1, D).astype(jnp.float32),
                    W2.astype(jnp.float32))
    acc2 = _agg_call(hp2, srcc, dstc, zrow)
    out = _tc3_call(deg, acc2, b2.reshape(1, D).astype(jnp.float32), batchp,
                    Wf.astype(jnp.float32), bf.reshape(1, DO).astype(jnp.float32))
    return out


# speedup vs baseline: 2.6549x; 2.6549x over previous
import functools

import jax
import jax.numpy as jnp
from jax import lax
from jax.experimental import pallas as pl
from jax.experimental.pallas import tpu as pltpu
from jax.experimental.pallas import tpu_sc as plsc

N = 10000
E = 320000
D = 128
G = 64
DO = 16

NC = 2
NS = 16
NW = NC * NS
LANE = 128
CH0 = 80
CH1 = 80
TOTCH = NS * (CH0 + CH1)
CHD = TOTCH // NW
PH = 2
CH0P = CH0 // PH
CH1P = CH1 // PH
CHPM = CH0P
PADCH = TOTCH + CHPM - CH1P
NBR = 2
SDEPTH = 8
EP = PADCH * LANE
NP = 10240
RPC = NP // NS
CW = D
BLK = 512
GRID = NP // BLK


def _deg_body(dst2, ones_hbm, zcnt_hbm, out, dstv, onesv, cnt_sh, ssem):
    c = lax.axis_index("c")
    s = lax.axis_index("s")
    w = s * NC + c
    sl = pl.ds(s * RPC, RPC)
    pltpu.sync_copy(zcnt_hbm.at[sl], cnt_sh.at[sl])
    pltpu.sync_copy(ones_hbm, onesv)
    pltpu.sync_copy(dst2.at[pl.ds(w * CHD, CHD)], dstv)
    plsc.subcore_barrier()

    def chunk(j, carry):
        pltpu.async_copy(onesv, cnt_sh.at[dstv.at[j]], ssem, add=True)

        @pl.when(j >= SDEPTH)
        def _():
            pltpu.make_async_copy(onesv, cnt_sh.at[dstv.at[0]], ssem).wait()

        return carry

    lax.fori_loop(0, CHD, chunk, 0)

    def drain(j, carry):
        pltpu.make_async_copy(onesv, cnt_sh.at[dstv.at[0]], ssem).wait()
        return carry

    lax.fori_loop(0, min(SDEPTH, CHD), drain, 0)
    plsc.subcore_barrier()
    pltpu.sync_copy(cnt_sh.at[sl], out.at[c, sl])


@functools.lru_cache(maxsize=None)
def _sc_calls():
    mesh = plsc.VectorSubcoreMesh(
        core_axis_name="c", subcore_axis_name="s",
        num_cores=NC, num_subcores=NS)
    deg_call = pl.kernel(
        _deg_body,
        out_type=jax.ShapeDtypeStruct((NC, NP, CW), jnp.float32),
        mesh=mesh,
        scratch_types=[
            pltpu.VMEM((CHD, LANE), jnp.int32),
            pltpu.VMEM((LANE, CW), jnp.float32),
            pltpu.VMEM_SHARED((NP, CW), jnp.float32),
            pltpu.SemaphoreType.DMA,
        ],
    )
    agg_call = pl.kernel(
        _agg_body,
        out_type=jax.ShapeDtypeStruct((NC, NP, D), jnp.float32),
        mesh=mesh,
        scratch_types=[
            pltpu.VMEM((CHPM, LANE), jnp.int32),
            pltpu.VMEM((CHPM, LANE), jnp.int32),
            pltpu.VMEM((NBR * LANE, D), jnp.float32),
            pltpu.VMEM_SHARED((NP, D), jnp.float32),
            pltpu.SemaphoreType.DMA,
        ],
    )
    return deg_call, agg_call


def _agg_body(hp, srcc, dstc, zrow_hbm, out, srcv, dstv, rows, acc_sh, gsem):
    c = lax.axis_index("c")
    s = lax.axis_index("s")
    sl = pl.ds(s * RPC, RPC)
    chnp = jnp.where(c == 0, CH0P, CH1P)
    start = jnp.where(c == 0, s * CH0, NS * CH0 + s * CH1)

    @pl.when(c == 0)
    def _():
        pltpu.sync_copy(hp.at[sl], acc_sh.at[sl])

    @pl.when(c != 0)
    def _():
        pltpu.sync_copy(zrow_hbm.at[sl], acc_sh.at[sl])

    plsc.subcore_barrier()

    def slot(j):
        return pl.ds((j % NBR) * LANE, LANE)

    for p in range(PH):
        base = start + p * chnp
        pltpu.sync_copy(srcc.at[pl.ds(base, CHPM)], srcv)
        pltpu.sync_copy(dstc.at[pl.ds(base, CHPM)], dstv)

        pltpu.async_copy(hp.at[srcv.at[0]], rows.at[slot(0)], gsem)

        def chunk(j, carry):
            bs = slot(j)
            pltpu.make_async_copy(hp.at[srcv.at[j]], rows.at[bs], gsem).wait()

            @pl.when(j + 1 < chnp)
            def _():
                pltpu.async_copy(hp.at[srcv.at[j + 1]], rows.at[slot(j + 1)],
                                 gsem)

            pltpu.sync_copy(rows.at[bs], acc_sh.at[dstv.at[j]], add=True)
            return carry

        lax.fori_loop(0, chnp, chunk, 0)

    plsc.subcore_barrier()
    pltpu.sync_copy(acc_sh.at[sl], out.at[c, sl])


def _dinv_from(deg_ref):
    cnt = jnp.max(deg_ref[0] + deg_ref[1], axis=1, keepdims=True)
    return lax.rsqrt(cnt + 1.0)


def _tc1_body(deg_ref, x_ref, w_ref, out_ref):
    dinv = _dinv_from(deg_ref)
    h = jnp.dot(x_ref[...], w_ref[...], preferred_element_type=jnp.float32)
    out_ref[...] = h * dinv


_tc1_call = pl.pallas_call(
    _tc1_body,
    grid=(GRID,),
    in_specs=[
        pl.BlockSpec((NC, BLK, CW), lambda i: (0, i, 0)),
        pl.BlockSpec((BLK, D), lambda i: (i, 0)),
        pl.BlockSpec((D, D), lambda i: (0, 0)),
    ],
    out_specs=pl.BlockSpec((BLK, D), lambda i: (i, 0)),
    out_shape=jax.ShapeDtypeStruct((NP, D), jnp.float32),
)


def _tc2_body(deg_ref, acc_ref, b_ref, w_ref, out_ref):
    dinv = _dinv_from(deg_ref)
    z = jnp.maximum(dinv * (acc_ref[0] + acc_ref[1]) + b_ref[...], 0.0)
    out_ref[...] = jnp.dot(
        z, w_ref[...], preferred_element_type=jnp.float32) * dinv


_tc2_call = pl.pallas_call(
    _tc2_body,
    grid=(GRID,),
    in_specs=[
        pl.BlockSpec((NC, BLK, CW), lambda i: (0, i, 0)),
        pl.BlockSpec((NC, BLK, D), lambda i: (0, i, 0)),
        pl.BlockSpec((1, D), lambda i: (0, 0)),
        pl.BlockSpec((D, D), lambda i: (0, 0)),
    ],
    out_specs=pl.BlockSpec((BLK, D), lambda i: (i, 0)),
    out_shape=jax.ShapeDtypeStruct((NP, D), jnp.float32),
)


def _tc3_body(deg_ref, acc_ref, b_ref, batch_ref, wf_ref, bf_ref, out_ref,
              pooled, cntp):
    i = pl.program_id(0)

    @pl.when(i == 0)
    def _():
        pooled[...] = jnp.zeros_like(pooled)
        cntp[...] = jnp.zeros_like(cntp)

    dinv = _dinv_from(deg_ref)
    z = jnp.maximum(dinv * (acc_ref[0] + acc_ref[1]) + b_ref[...], 0.0)
    oh = (batch_ref[...] == lax.broadcasted_iota(
        jnp.int32, (BLK, G), 1)).astype(jnp.float32)
    dn = (((0,), (0,)), ((), ()))
    pooled[...] += lax.dot_general(oh, z, dn,
                                   preferred_element_type=jnp.float32)
    cntp[...] += lax.dot_general(oh, jnp.ones((BLK, D), jnp.float32), dn,
                                 preferred_element_type=jnp.float32)
    pool = pooled[...] / jnp.maximum(cntp[...], 1.0)
    out_ref[...] = jnp.dot(
        pool, wf_ref[...], preferred_element_type=jnp.float32) + bf_ref[...]


_tc3_call = pl.pallas_call(
    _tc3_body,
    grid=(GRID,),
    in_specs=[
        pl.BlockSpec((NC, BLK, CW), lambda i: (0, i, 0)),
        pl.BlockSpec((NC, BLK, D), lambda i: (0, i, 0)),
        pl.BlockSpec((1, D), lambda i: (0, 0)),
        pl.BlockSpec((BLK, 1), lambda i: (i, 0)),
        pl.BlockSpec((D, DO), lambda i: (0, 0)),
        pl.BlockSpec((1, DO), lambda i: (0, 0)),
    ],
    out_specs=pl.BlockSpec((G, DO), lambda i: (0, 0)),
    out_shape=jax.ShapeDtypeStruct((G, DO), jnp.float32),
    scratch_shapes=[
        pltpu.VMEM((G, D), jnp.float32),
        pltpu.VMEM((G, D), jnp.float32),
    ],
)


def kernel(x, edge_index, batch, W1, b1, W2, b2, Wf, bf):
    i32 = jnp.int32
    pad_idx = N + (jnp.arange(EP - E, dtype=i32) % (NP - N))
    srcc = jnp.concatenate([edge_index[0].astype(i32), pad_idx])
    srcc = srcc.reshape(PADCH, LANE)
    dstc = jnp.concatenate([edge_index[1].astype(i32), pad_idx])
    dstc = dstc.reshape(PADCH, LANE)
    xp = jnp.pad(x.astype(jnp.float32), ((0, NP - N), (0, 0)))
    batchp = jnp.pad(batch.astype(i32), (0, NP - N),
                     constant_values=G).reshape(NP, 1)
    ones_cw = jnp.ones((LANE, CW), jnp.float32)
    zrow = jnp.zeros((NP, D), jnp.float32)

    _deg_call, _agg_call = _sc_calls()
    deg = _deg_call(dstc, ones_cw, zrow)
    hp1 = _tc1_call(deg, xp, W1.astype(jnp.float32))
    acc1 = _agg_call(hp1, srcc, dstc, zrow)
    hp2 = _tc2_call(deg, acc1, b1.reshape(1, D).astype(jnp.float32),
                    W2.astype(jnp.float32))
    acc2 = _agg_call(hp2, srcc, dstc, zrow)
    out = _tc3_call(deg, acc2, b2.reshape(1, D).astype(jnp.float32), batchp,
                    Wf.astype(jnp.float32), bf.reshape(1, DO).astype(jnp.float32))
    return out


# speedup vs baseline: 2.6590x; 1.0015x over previous
import functools

import jax
import jax.numpy as jnp
from jax import lax
from jax.experimental import pallas as pl
from jax.experimental.pallas import tpu as pltpu
from jax.experimental.pallas import tpu_sc as plsc

N = 10000
E = 320000
D = 128
G = 64
DO = 16

NC = 2
NS = 16
NW = NC * NS
LANE = 128
CH0 = 80
CH1 = 80
TOTCH = NS * (CH0 + CH1)
CHD = TOTCH // NW
PH = 2
CH0P = CH0 // PH
CH1P = CH1 // PH
CHPM = CH0P
PADCH = TOTCH + CHPM - CH1P
NBR = 2
SDEPTH = 8
EP = PADCH * LANE
NP = 10240
RPC = NP // NS
CW = D
BLK = 512
GRID = NP // BLK


def _deg_body(dst2, ones_hbm, zcnt_hbm, out, dstv, onesv, cnt_sh, ssem):
    c = lax.axis_index("c")
    s = lax.axis_index("s")
    w = s * NC + c
    sl = pl.ds(s * RPC, RPC)
    pltpu.sync_copy(zcnt_hbm.at[sl], cnt_sh.at[sl])
    pltpu.sync_copy(ones_hbm, onesv)
    pltpu.sync_copy(dst2.at[pl.ds(w * CHD, CHD)], dstv)
    plsc.subcore_barrier()

    def chunk(j, carry):
        pltpu.async_copy(onesv, cnt_sh.at[dstv.at[j]], ssem, add=True)

        @pl.when(j >= SDEPTH)
        def _():
            pltpu.make_async_copy(onesv, cnt_sh.at[dstv.at[0]], ssem).wait()

        return carry

    lax.fori_loop(0, CHD, chunk, 0)

    def drain(j, carry):
        pltpu.make_async_copy(onesv, cnt_sh.at[dstv.at[0]], ssem).wait()
        return carry

    lax.fori_loop(0, min(SDEPTH, CHD), drain, 0)
    plsc.subcore_barrier()
    pltpu.sync_copy(cnt_sh.at[sl], out.at[c, sl])


@functools.lru_cache(maxsize=None)
def _sc_calls():
    mesh = plsc.VectorSubcoreMesh(
        core_axis_name="c", subcore_axis_name="s",
        num_cores=NC, num_subcores=NS)
    deg_call = pl.kernel(
        _deg_body,
        out_type=jax.ShapeDtypeStruct((NC, NP, CW), jnp.float32),
        mesh=mesh,
        scratch_types=[
            pltpu.VMEM((CHD, LANE), jnp.int32),
            pltpu.VMEM((LANE, CW), jnp.float32),
            pltpu.VMEM_SHARED((NP, CW), jnp.float32),
            pltpu.SemaphoreType.DMA,
        ],
    )
    agg_call = pl.kernel(
        _agg_body,
        out_type=jax.ShapeDtypeStruct((NC, NP, D), jnp.float32),
        mesh=mesh,
        scratch_types=[
            pltpu.VMEM((CHPM, LANE), jnp.int32),
            pltpu.VMEM((CHPM, LANE), jnp.int32),
            pltpu.VMEM((NBR * LANE, D), jnp.float32),
            pltpu.VMEM_SHARED((NP, D), jnp.float32),
            pltpu.SemaphoreType.DMA,
        ],
    )
    return deg_call, agg_call


def _agg_body(hp, srcc, dstc, zrow_hbm, out, srcv, dstv, rows, acc_sh, gsem):
    c = lax.axis_index("c")
    s = lax.axis_index("s")
    sl = pl.ds(s * RPC, RPC)
    chnp = jnp.where(c == 0, CH0P, CH1P)
    start = jnp.where(c == 0, s * CH0, NS * CH0 + s * CH1)

    @pl.when(c == 0)
    def _():
        pltpu.sync_copy(hp.at[sl], acc_sh.at[sl])

    @pl.when(c != 0)
    def _():
        pltpu.sync_copy(zrow_hbm.at[sl], acc_sh.at[sl])

    plsc.subcore_barrier()

    def slot(j):
        return pl.ds((j % NBR) * LANE, LANE)

    for p in range(PH):
        base = start + p * chnp
        pltpu.sync_copy(srcc.at[pl.ds(base, CHPM)], srcv)
        pltpu.sync_copy(dstc.at[pl.ds(base, CHPM)], dstv)

        pltpu.async_copy(hp.at[srcv.at[0]], rows.at[slot(0)], gsem)

        def chunk(j, carry):
            bs = slot(j)
            pltpu.make_async_copy(hp.at[srcv.at[j]], rows.at[bs], gsem).wait()

            @pl.when(j + 1 < chnp)
            def _():
                pltpu.async_copy(hp.at[srcv.at[j + 1]], rows.at[slot(j + 1)],
                                 gsem)

            pltpu.sync_copy(rows.at[bs], acc_sh.at[dstv.at[j]], add=True)
            return carry

        lax.fori_loop(0, chnp, chunk, 0)

    plsc.subcore_barrier()
    pltpu.sync_copy(acc_sh.at[sl], out.at[c, sl])


def _dinv_from(deg_ref):
    cnt = jnp.max(deg_ref[0] + deg_ref[1], axis=1, keepdims=True)
    return lax.rsqrt(cnt + 1.0)


def _tc1a_body(x_ref, w_ref, out_ref):
    out_ref[...] = jnp.dot(x_ref[...], w_ref[...],
                           preferred_element_type=jnp.float32)


_tc1a_call = pl.pallas_call(
    _tc1a_body,
    grid=(GRID,),
    in_specs=[
        pl.BlockSpec((BLK, D), lambda i: (i, 0)),
        pl.BlockSpec((D, D), lambda i: (0, 0)),
    ],
    out_specs=pl.BlockSpec((BLK, D), lambda i: (i, 0)),
    out_shape=jax.ShapeDtypeStruct((NP, D), jnp.float32),
)


def _tc1b_body(deg_ref, h_ref, out_ref):
    out_ref[...] = h_ref[...] * _dinv_from(deg_ref)


_tc1b_call = pl.pallas_call(
    _tc1b_body,
    grid=(GRID,),
    in_specs=[
        pl.BlockSpec((NC, BLK, CW), lambda i: (0, i, 0)),
        pl.BlockSpec((BLK, D), lambda i: (i, 0)),
    ],
    out_specs=pl.BlockSpec((BLK, D), lambda i: (i, 0)),
    out_shape=jax.ShapeDtypeStruct((NP, D), jnp.float32),
)


def _tc2_body(deg_ref, acc_ref, b_ref, w_ref, out_ref):
    dinv = _dinv_from(deg_ref)
    z = jnp.maximum(dinv * (acc_ref[0] + acc_ref[1]) + b_ref[...], 0.0)
    out_ref[...] = jnp.dot(
        z, w_ref[...], preferred_element_type=jnp.float32) * dinv


_tc2_call = pl.pallas_call(
    _tc2_body,
    grid=(GRID,),
    in_specs=[
        pl.BlockSpec((NC, BLK, CW), lambda i: (0, i, 0)),
        pl.BlockSpec((NC, BLK, D), lambda i: (0, i, 0)),
        pl.BlockSpec((1, D), lambda i: (0, 0)),
        pl.BlockSpec((D, D), lambda i: (0, 0)),
    ],
    out_specs=pl.BlockSpec((BLK, D), lambda i: (i, 0)),
    out_shape=jax.ShapeDtypeStruct((NP, D), jnp.float32),
)


def _tc3_body(deg_ref, acc_ref, b_ref, batch_ref, wf_ref, bf_ref, out_ref,
              pooled, cntp):
    i = pl.program_id(0)

    @pl.when(i == 0)
    def _():
        pooled[...] = jnp.zeros_like(pooled)
        cntp[...] = jnp.zeros_like(cntp)

    dinv = _dinv_from(deg_ref)
    z = jnp.maximum(dinv * (acc_ref[0] + acc_ref[1]) + b_ref[...], 0.0)
    oh = (batch_ref[...] == lax.broadcasted_iota(
        jnp.int32, (BLK, G), 1)).astype(jnp.float32)
    dn = (((0,), (0,)), ((), ()))
    pooled[...] += lax.dot_general(oh, z, dn,
                                   preferred_element_type=jnp.float32)
    cntp[...] += lax.dot_general(oh, jnp.ones((BLK, D), jnp.float32), dn,
                                 preferred_element_type=jnp.float32)
    pool = pooled[...] / jnp.maximum(cntp[...], 1.0)
    out_ref[...] = jnp.dot(
        pool, wf_ref[...], preferred_element_type=jnp.float32) + bf_ref[...]


_tc3_call = pl.pallas_call(
    _tc3_body,
    grid=(GRID,),
    in_specs=[
        pl.BlockSpec((NC, BLK, CW), lambda i: (0, i, 0)),
        pl.BlockSpec((NC, BLK, D), lambda i: (0, i, 0)),
        pl.BlockSpec((1, D), lambda i: (0, 0)),
        pl.BlockSpec((BLK, 1), lambda i: (i, 0)),
        pl.BlockSpec((D, DO), lambda i: (0, 0)),
        pl.BlockSpec((1, DO), lambda i: (0, 0)),
    ],
    out_specs=pl.BlockSpec((G, DO), lambda i: (0, 0)),
    out_shape=jax.ShapeDtypeStruct((G, DO), jnp.float32),
    scratch_shapes=[
        pltpu.VMEM((G, D), jnp.float32),
        pltpu.VMEM((G, D), jnp.float32),
    ],
)


def kernel(x, edge_index, batch, W1, b1, W2, b2, Wf, bf):
    i32 = jnp.int32
    pad_idx = N + (jnp.arange(EP - E, dtype=i32) % (NP - N))
    srcc = jnp.concatenate([edge_index[0].astype(i32), pad_idx])
    srcc = srcc.reshape(PADCH, LANE)
    dstc = jnp.concatenate([edge_index[1].astype(i32), pad_idx])
    dstc = dstc.reshape(PADCH, LANE)
    xp = jnp.pad(x.astype(jnp.float32), ((0, NP - N), (0, 0)))
    batchp = jnp.pad(batch.astype(i32), (0, NP - N),
                     constant_values=G).reshape(NP, 1)
    ones_cw = jnp.ones((LANE, CW), jnp.float32)
    zrow = jnp.zeros((NP, D), jnp.float32)

    _deg_call, _agg_call = _sc_calls()
    deg = _deg_call(dstc, ones_cw, zrow)
    h1 = _tc1a_call(xp, W1.astype(jnp.float32))
    hp1 = _tc1b_call(deg, h1)
    acc1 = _agg_call(hp1, srcc, dstc, zrow)
    hp2 = _tc2_call(deg, acc1, b1.reshape(1, D).astype(jnp.float32),
                    W2.astype(jnp.float32))
    acc2 = _agg_call(hp2, srcc, dstc, zrow)
    out = _tc3_call(deg, acc2, b2.reshape(1, D).astype(jnp.float32), batchp,
                    Wf.astype(jnp.float32), bf.reshape(1, DO).astype(jnp.float32))
    return out
